# baseline (speedup 1.0000x reference)
import functools

import jax
import jax.numpy as jnp
from jax.experimental import pallas as pl


def _matmul_body(x_ref, w_ref, o_ref):
    o_ref[...] = jnp.dot(x_ref[...], w_ref[...],
                         preferred_element_type=jnp.float32)


def _layer1_body(adj_ref, s0_ref, b0_ref, w1_ref, o_ref):
    h = jnp.dot(adj_ref[...], s0_ref[...], preferred_element_type=jnp.float32)
    h = jnp.maximum(h + b0_ref[...], 0.0)
    o_ref[...] = jnp.dot(h, w1_ref[...], preferred_element_type=jnp.float32)


def _layer2_body(adj_ref, s1_ref, b1_ref, o_ref):
    h = jnp.dot(adj_ref[...], s1_ref[...], preferred_element_type=jnp.float32)
    o_ref[...] = jnp.maximum(h + b1_ref[...], 0.0)


def _pick_bm(n):
    for bm in (400, 200, 100, 40, 8):
        if n % bm == 0:
            return bm
    return n


@functools.partial(jax.jit, static_argnames=())
def kernel(x, adj, W0, b0, W1, b1):
    n, in_ch = x.shape
    hid = W0.shape[1]
    out_ch = W1.shape[1]
    bm = _pick_bm(n)
    grid = n // bm

    s0 = pl.pallas_call(
        _matmul_body,
        out_shape=jax.ShapeDtypeStruct((n, hid), jnp.float32),
    )(x, W0)

    b0r = b0.reshape(1, hid)
    b1r = b1.reshape(1, out_ch)

    s1 = pl.pallas_call(
        _layer1_body,
        grid=(grid,),
        in_specs=[
            pl.BlockSpec((bm, n), lambda i: (i, 0)),
            pl.BlockSpec((n, hid), lambda i: (0, 0)),
            pl.BlockSpec((1, hid), lambda i: (0, 0)),
            pl.BlockSpec((hid, out_ch), lambda i: (0, 0)),
        ],
        out_specs=pl.BlockSpec((bm, out_ch), lambda i: (i, 0)),
        out_shape=jax.ShapeDtypeStruct((n, out_ch), jnp.float32),
    )(adj, s0, b0r, W1)

    out = pl.pallas_call(
        _layer2_body,
        grid=(grid,),
        in_specs=[
            pl.BlockSpec((bm, n), lambda i: (i, 0)),
            pl.BlockSpec((n, out_ch), lambda i: (0, 0)),
            pl.BlockSpec((1, out_ch), lambda i: (0, 0)),
        ],
        out_specs=pl.BlockSpec((bm, out_ch), lambda i: (i, 0)),
        out_shape=jax.ShapeDtypeStruct((n, out_ch), jnp.float32),
    )(adj, s1, b1r)
    return out


# speedup vs baseline: 1.0538x; 1.0538x over previous
import functools

import jax
import jax.numpy as jnp
from jax.experimental import pallas as pl
from jax.experimental.pallas import tpu as pltpu


def _body(x_ref, adj_ref, w0_ref, b0_ref, w1_ref, b1_ref, out_ref,
          s0_ref, s1_ref, *, bm):
    p = pl.program_id(0)
    i = pl.program_id(1)

    @pl.when(jnp.logical_and(p == 0, i == 0))
    def _():
        s0_ref[...] = jnp.dot(x_ref[...], w0_ref[...],
                              preferred_element_type=jnp.float32)

    @pl.when(p == 0)
    def _():
        h = jnp.dot(adj_ref[...], s0_ref[...],
                    preferred_element_type=jnp.float32)
        h = jnp.maximum(h + b0_ref[...], 0.0)
        s1_ref[pl.ds(i * bm, bm), :] = jnp.dot(
            h, w1_ref[...], preferred_element_type=jnp.float32)

    @pl.when(p == 1)
    def _():
        h = jnp.dot(adj_ref[...], s1_ref[...],
                    preferred_element_type=jnp.float32)
        out_ref[...] = jnp.maximum(h + b1_ref[...], 0.0)


def _pick_bm(n):
    for bm in (400, 200, 100, 40, 8):
        if n % bm == 0:
            return bm
    return n


@jax.jit
def kernel(x, adj, W0, b0, W1, b1):
    n, in_ch = x.shape
    hid = W0.shape[1]
    out_ch = W1.shape[1]
    bm = _pick_bm(n)
    nblk = n // bm

    b0r = b0.reshape(1, hid)
    b1r = b1.reshape(1, out_ch)

    out = pl.pallas_call(
        functools.partial(_body, bm=bm),
        grid=(2, nblk),
        in_specs=[
            pl.BlockSpec((n, in_ch), lambda p, i: (0, 0)),
            pl.BlockSpec((bm, n), lambda p, i: (i, 0)),
            pl.BlockSpec((in_ch, hid), lambda p, i: (0, 0)),
            pl.BlockSpec((1, hid), lambda p, i: (0, 0)),
            pl.BlockSpec((hid, out_ch), lambda p, i: (0, 0)),
            pl.BlockSpec((1, out_ch), lambda p, i: (0, 0)),
        ],
        out_specs=pl.BlockSpec((bm, out_ch), lambda p, i: (i * p, 0)),
        out_shape=jax.ShapeDtypeStruct((n, out_ch), jnp.float32),
        scratch_shapes=[
            pltpu.VMEM((n, hid), jnp.float32),
            pltpu.VMEM((n, out_ch), jnp.float32),
        ],
    )(x, adj, W0, b0r, W1, b1r)
    return out
